# baseline (speedup 1.0000x reference)
import functools
import math

import numpy as np
import jax
import jax.numpy as jnp
from jax import lax
from jax.experimental import pallas as pl
from jax.experimental.pallas import tpu as pltpu
from jax.experimental.pallas import tpu_sc as plsc

_VOCAB = 100000
_D = 1024
_B = 4
_L = 2048
_NC, _NS = 2, 16
_NW = _NC * _NS
_ROWS = _B * _L
_RPW = _ROWS // _NW
_CHUNK = 16
_NCHUNK = _RPW // _CHUNK
_LANES = 16
_UNROLL = 8


def _pos_encoding(seq_len: int, dim: int) -> np.ndarray:
    pe = np.zeros((seq_len, dim), dtype=np.float32)
    position = np.arange(0, seq_len, dtype=np.float32)[:, None]
    div_term = np.exp(
        np.arange(0, dim, 2).astype(np.float32) * (-math.log(10000.0) / dim))
    pe[:, 0::2] = np.sin(position * div_term)
    pe[:, 1::2] = np.cos(position * div_term)
    return pe


_PE = _pos_encoding(_L, _D)


def _embed_body(x_hbm, pe_hbm, table_hbm, out_hbm, idx_v, gbuf, pbuf, sem):
    wid = lax.axis_index("s") * _NC + lax.axis_index("c")
    base = wid * _RPW
    pos_base = lax.rem(base, _L)

    pltpu.sync_copy(x_hbm.at[pl.ds(base, _RPW)], idx_v)

    def chunk_body(c, carry):
        row0 = c * _CHUNK
        pltpu.async_copy(
            table_hbm.at[idx_v.at[pl.ds(row0, _CHUNK)]], gbuf, sem).wait()
        pltpu.sync_copy(pe_hbm.at[pl.ds(pos_base + row0, _CHUNK)], pbuf)

        def row_body(r, carry2):
            def col_body(j, carry3):
                for u in range(_UNROLL):
                    s = pl.ds(j * (_LANES * _UNROLL) + u * _LANES, _LANES)
                    gbuf[r, s] = gbuf[r, s] + pbuf[r, s]
                return carry3
            return lax.fori_loop(0, _D // (_LANES * _UNROLL), col_body, carry2)

        lax.fori_loop(0, _CHUNK, row_body, 0)
        pltpu.sync_copy(gbuf, out_hbm.at[pl.ds(base + row0, _CHUNK)])
        return carry

    lax.fori_loop(0, _NCHUNK, chunk_body, 0)


@jax.jit
def _embed(x_flat, pe, table):
    mesh = plsc.VectorSubcoreMesh(core_axis_name="c", subcore_axis_name="s")
    f = functools.partial(
        pl.kernel,
        mesh=mesh,
        out_type=jax.ShapeDtypeStruct((_ROWS, _D), jnp.float32),
        scratch_types=[
            pltpu.VMEM((_RPW,), jnp.int32),
            pltpu.VMEM((_CHUNK, _D), jnp.float32),
            pltpu.VMEM((_CHUNK, _D), jnp.float32),
            pltpu.SemaphoreType.DMA,
        ],
    )(_embed_body)
    return f(x_flat, pe, table)


def kernel(x, table):
    pe = jnp.asarray(_PE)
    out = _embed(x.reshape(_ROWS), pe, table)
    return out.reshape(_B, _L, _D)


# speedup vs baseline: 3.0167x; 3.0167x over previous
import functools
import math

import numpy as np
import jax
import jax.numpy as jnp
from jax import lax
from jax.experimental import pallas as pl
from jax.experimental.pallas import tpu as pltpu
from jax.experimental.pallas import tpu_sc as plsc

_VOCAB = 100000
_D = 1024
_B = 4
_L = 2048
_NC, _NS = 2, 16
_NW = _NC * _NS
_PPW = _L // _NW
_ROWS = _B * _L
_CHUNK = 16
_NGROUP = _PPW // _CHUNK
_NCHUNK = _NGROUP * _B
_LANES = 16
_VPC = _CHUNK * _D // _LANES


def _pos_encoding(seq_len: int, dim: int) -> np.ndarray:
    pe = np.zeros((seq_len, dim), dtype=np.float32)
    position = np.arange(0, seq_len, dtype=np.float32)[:, None]
    div_term = np.exp(
        np.arange(0, dim, 2).astype(np.float32) * (-math.log(10000.0) / dim))
    pe[:, 0::2] = np.sin(position * div_term)
    pe[:, 1::2] = np.cos(position * div_term)
    return pe


_PE = _pos_encoding(_L, _D)


def _embed_body(x_hbm, pe_hbm, table_hbm, out_hbm,
                idx_v, pe0, pe1, gb0, gb1, ob0, ob1,
                psem0, psem1, gsem0, gsem1, ssem0, ssem1):
    pe_v = (pe0, pe1)
    gbuf = (gb0, gb1)
    obuf = (ob0, ob1)
    psem = (psem0, psem1)
    gsem = (gsem0, gsem1)
    ssem = (ssem0, ssem1)

    wid = lax.axis_index("s") * _NC + lax.axis_index("c")
    p0 = wid * _PPW

    for b in range(_B):
        pltpu.sync_copy(x_hbm.at[b, pl.ds(p0, _PPW)],
                        idx_v.at[pl.ds(b * _PPW, _PPW)])

    def start_gather(c):
        g, bt = divmod(c, _B)
        cp = pltpu.async_copy(
            table_hbm.at[idx_v.at[pl.ds(bt * _PPW + g * _CHUNK, _CHUNK)]],
            gbuf[c % 2], gsem[c % 2])
        return cp

    def start_pe(g):
        return pltpu.async_copy(
            pe_hbm.at[pl.ds(p0 + g * _CHUNK, _CHUNK)], pe_v[g % 2],
            psem[g % 2])

    gathers = {0: start_gather(0), 1: start_gather(1)}
    pes = {0: start_pe(0), 1: start_pe(1)}
    stores = {}

    for c in range(_NCHUNK):
        g, bt = divmod(c, _B)
        if bt == 0 and 1 <= g <= _NGROUP - 2:
            pes[g + 1] = start_pe(g + 1)
        gathers.pop(c).wait()
        if c >= 2:
            stores.pop(c - 2).wait()
        if bt == 0:
            pes.pop(g).wait()

        gb, ob, pb = gbuf[c % 2], obuf[c % 2], pe_v[g % 2]

        @plsc.parallel_loop(0, _VPC, 1, unroll=8)
        def add_body(i):
            r = lax.shift_right_logical(i, 6)
            col = lax.bitwise_and(i, _D // _LANES - 1) * _LANES
            s = pl.ds(pl.multiple_of(col, _LANES), _LANES)
            ob[r, s] = gb[r, s] + pb[r, s]

        stores[c] = pltpu.async_copy(
            ob, out_hbm.at[pl.ds(bt * _L + p0 + g * _CHUNK, _CHUNK)],
            ssem[c % 2])
        if c + 2 < _NCHUNK:
            gathers[c + 2] = start_gather(c + 2)

    stores.pop(_NCHUNK - 2).wait()
    stores.pop(_NCHUNK - 1).wait()


@jax.jit
def _embed(x, pe, table):
    mesh = plsc.VectorSubcoreMesh(core_axis_name="c", subcore_axis_name="s")
    f = functools.partial(
        pl.kernel,
        mesh=mesh,
        out_type=jax.ShapeDtypeStruct((_ROWS, _D), jnp.float32),
        scratch_types=[
            pltpu.VMEM((_B * _PPW,), jnp.int32),
            pltpu.VMEM((_CHUNK, _D), jnp.float32),
            pltpu.VMEM((_CHUNK, _D), jnp.float32),
            pltpu.VMEM((_CHUNK, _D), jnp.float32),
            pltpu.VMEM((_CHUNK, _D), jnp.float32),
            pltpu.VMEM((_CHUNK, _D), jnp.float32),
            pltpu.VMEM((_CHUNK, _D), jnp.float32),
            pltpu.SemaphoreType.DMA,
            pltpu.SemaphoreType.DMA,
            pltpu.SemaphoreType.DMA,
            pltpu.SemaphoreType.DMA,
            pltpu.SemaphoreType.DMA,
            pltpu.SemaphoreType.DMA,
        ],
    )(_embed_body)
    return f(x, pe, table)


def kernel(x, table):
    pe = jnp.asarray(_PE)
    out = _embed(x, pe, table)
    return out.reshape(_B, _L, _D)
